# baseline (speedup 1.0000x reference)
import functools

import jax
import jax.numpy as jnp
from jax import lax
from jax.experimental import pallas as pl
from jax.experimental.pallas import tpu as pltpu
from jax.experimental.pallas import tpu_sc as plsc

N = 16384
EMB = 64
NNZ = 268435
LAYERS = 3

NUM_CORES = 2
NUM_SUBCORES = 16
NUM_WORKERS = NUM_CORES * NUM_SUBCORES
CHUNK = 96
CHUNKS_PER_WORKER = 88
EDGES_PER_WORKER = CHUNK * CHUNKS_PER_WORKER
NNZ_PAD = NUM_WORKERS * EDGES_PER_WORKER
ROWS_PER_TILE = N // NUM_SUBCORES
NGBUF = 6
NSBUF = 3
MROWS = N // NUM_WORKERS
HALF = N // NUM_CORES
MCHUNKS = [CHUNK] * (MROWS // CHUNK) + ([MROWS % CHUNK] if MROWS % CHUNK else [])

_mesh = plsc.VectorSubcoreMesh(core_axis_name="c", subcore_axis_name="s")

_PK = plsc.PackFormat.INTERLEAVED


@functools.partial(
    pl.kernel,
    out_type=[
        jax.ShapeDtypeStruct((N, EMB), jnp.float32),
        jax.ShapeDtypeStruct((N, EMB), jnp.bfloat16),
        jax.ShapeDtypeStruct((N, EMB), jnp.float32),
        jax.ShapeDtypeStruct((NUM_CORES, HALF, EMB), jnp.float32),
    ],
    mesh=_mesh,
    compiler_params=pltpu.CompilerParams(
        use_tc_tiling_on_sc=False, needs_layout_passes=False
    ),
    scratch_types=[
        pltpu.VMEM((CHUNKS_PER_WORKER, CHUNK), jnp.int32),
        pltpu.VMEM((CHUNKS_PER_WORKER, CHUNK), jnp.int32),
        pltpu.VMEM((CHUNKS_PER_WORKER, CHUNK), jnp.float32),
        pltpu.VMEM((NGBUF, CHUNK, EMB), jnp.bfloat16),
        pltpu.VMEM((NSBUF, CHUNK, EMB), jnp.float32),
        pltpu.VMEM_SHARED((N, EMB), jnp.float32),
        pltpu.SemaphoreType.DMA((NGBUF,)),
        pltpu.SemaphoreType.DMA((NSBUF,)),
        pltpu.SemaphoreType.DMA((4,)),
        pltpu.SemaphoreType.REGULAR,
    ],
)
def _hyperconv_sc(rows_hbm, cols_hbm, vals_hbm, emb_hbm, zeros_hbm,
                  out_hbm, x_hbm, acc_hbm, p_hbm,
                  cols_v, rowi_v, vals_vm, gbuf, sbuf, partial,
                  gsem, ssem, msem, bar_sem):
    c = lax.axis_index("c")
    s = lax.axis_index("s")
    wid = c * NUM_SUBCORES + s
    mbase = wid * MROWS

    def global_barrier():
        plsc.subcore_barrier()

        @pl.when(s == 0)
        def _():
            pl.semaphore_signal(bar_sem, 1, core_index=1 - c)
            pl.semaphore_wait(bar_sem, 1)

        plsc.subcore_barrier()

    def pack_row_to(dst_ref, i, vecs):
        dst_ref[i, pl.ds(0, 32)] = plsc.pack(vecs[0], vecs[1], format=_PK)
        dst_ref[i, pl.ds(32, 32)] = plsc.pack(vecs[2], vecs[3], format=_PK)

    pltpu.sync_copy(cols_hbm.at[wid], cols_v)
    pltpu.sync_copy(rows_hbm.at[wid], rowi_v)
    pltpu.sync_copy(vals_hbm.at[wid], vals_vm)

    pltpu.sync_copy(zeros_hbm, partial.at[pl.ds(s * ROWS_PER_TILE, ROWS_PER_TILE)])
    off = 0
    for mlen in MCHUNKS:
        rs = mbase + off
        fsl = sbuf.at[0] if mlen == CHUNK else sbuf.at[0, pl.ds(0, mlen)]
        xsl = gbuf.at[0] if mlen == CHUNK else gbuf.at[0, pl.ds(0, mlen)]
        pltpu.sync_copy(emb_hbm.at[pl.ds(rs, mlen)], fsl)

        def irow(i, carry0):
            vecs = [sbuf[0, i, pl.ds(q * 16, 16)] for q in range(4)]
            pack_row_to(gbuf.at[0], i, vecs)
            return carry0

        lax.fori_loop(0, mlen, irow, 0)
        pltpu.sync_copy(xsl, x_hbm.at[pl.ds(rs, mlen)])
        pltpu.sync_copy(fsl, acc_hbm.at[pl.ds(rs, mlen)])
        off += mlen
    global_barrier()

    def layer_body(layer, carry):
        for b in range(NGBUF):
            pltpu.async_copy(x_hbm.at[cols_v.at[b]], gbuf.at[b], gsem.at[b])

        def step(j, b, p, refill, sprev):
            pltpu.make_async_copy(
                x_hbm.at[cols_v.at[j]], gbuf.at[b], gsem.at[b]
            ).wait()

            @pl.when(sprev)
            def _():
                pltpu.make_async_copy(
                    sbuf.at[p], partial.at[rowi_v.at[j]], ssem.at[p]
                ).wait()

            for g in range(CHUNK // 16):
                vv = vals_vm[j, pl.ds(g * 16, 16)]
                for k in range(16):
                    e = g * 16 + k
                    v = vv[k]
                    a0, a1 = plsc.unpack(gbuf[b, e, pl.ds(0, 32)], format=_PK)
                    a2, a3 = plsc.unpack(gbuf[b, e, pl.ds(32, 32)], format=_PK)
                    sbuf[p, e, pl.ds(0, 16)] = a0 * v
                    sbuf[p, e, pl.ds(16, 16)] = a1 * v
                    sbuf[p, e, pl.ds(32, 16)] = a2 * v
                    sbuf[p, e, pl.ds(48, 16)] = a3 * v

            @pl.when(refill)
            def _():
                pltpu.async_copy(
                    x_hbm.at[cols_v.at[j + NGBUF]], gbuf.at[b], gsem.at[b]
                )

            pltpu.async_copy(
                sbuf.at[p], partial.at[rowi_v.at[j]], ssem.at[p], add=True
            )

        n_outer = CHUNKS_PER_WORKER // NGBUF

        def outer_body(o, carry2):
            for b in range(NGBUF):
                j = o * NGBUF + b
                p = b % NSBUF if NGBUF % NSBUF == 0 else None
                assert p is not None
                refill = j + NGBUF < jnp.int32(CHUNKS_PER_WORKER)
                sprev = jnp.bool_(True) if b >= NSBUF else (o > 0)
                step(j, b, p, refill, sprev)
            return carry2

        lax.fori_loop(0, n_outer, outer_body, 0)
        for t in range(CHUNKS_PER_WORKER - n_outer * NGBUF):
            j = n_outer * NGBUF + t
            b = j % NGBUF
            p = j % NSBUF
            step(j, b, p, jnp.bool_(False), jnp.bool_(True))

        for p in range(NSBUF):
            pltpu.make_async_copy(
                sbuf.at[p], partial.at[rowi_v.at[0]], ssem.at[p]
            ).wait()
        plsc.subcore_barrier()

        exp_base = (1 - c) * HALF + s * MROWS
        pltpu.sync_copy(
            partial.at[pl.ds(exp_base, MROWS)],
            p_hbm.at[c, pl.ds(s * MROWS, MROWS)],
        )
        global_barrier()

        moff = 0
        pending = None
        for mi, mlen in enumerate(MCHUNKS):
            rs = mbase + moff
            ps = mbase - c * HALF + moff
            Asl = sbuf.at[0] if mlen == CHUNK else sbuf.at[0, pl.ds(0, mlen)]
            Bsl = sbuf.at[1] if mlen == CHUNK else sbuf.at[1, pl.ds(0, mlen)]
            Csl = sbuf.at[2] if mlen == CHUNK else sbuf.at[2, pl.ds(0, mlen)]
            Xsl = gbuf.at[0] if mlen == CHUNK else gbuf.at[0, pl.ds(0, mlen)]
            src_a = partial.at[pl.ds(rs, mlen)]
            src_b = p_hbm.at[1 - c, pl.ds(ps, mlen)]
            src_c = acc_hbm.at[pl.ds(rs, mlen)]
            if pending is not None:
                x_src, x_dst, a_src, a_dst = pending
                pltpu.make_async_copy(x_src, x_dst, msem.at[2]).wait()
                pltpu.make_async_copy(a_src, a_dst, msem.at[3]).wait()
            pltpu.async_copy(src_a, Asl, msem.at[0])
            pltpu.async_copy(src_b, Bsl, msem.at[1])
            pltpu.async_copy(src_c, Csl, msem.at[2])
            pltpu.make_async_copy(src_a, Asl, msem.at[0]).wait()
            pltpu.make_async_copy(src_b, Bsl, msem.at[1]).wait()
            pltpu.make_async_copy(src_c, Csl, msem.at[2]).wait()

            def mrow(i, carry3):
                xs = []
                for q in range(EMB // 16):
                    sl = pl.ds(q * 16, 16)
                    x = sbuf[0, i, sl] + sbuf[1, i, sl]
                    xs.append(x)
                    sbuf[2, i, sl] = sbuf[2, i, sl] + x
                pack_row_to(gbuf.at[0], i, xs)
                return carry3

            lax.fori_loop(0, mlen, mrow, 0)
            dst_x = x_hbm.at[pl.ds(rs, mlen)]
            dst_a = acc_hbm.at[pl.ds(rs, mlen)]
            pltpu.async_copy(Xsl, dst_x, msem.at[2])
            pltpu.async_copy(Csl, dst_a, msem.at[3])
            pending = (Xsl, dst_x, Csl, dst_a)
            moff += mlen
        if pending is not None:
            x_src, x_dst, a_src, a_dst = pending
            pltpu.make_async_copy(x_src, x_dst, msem.at[2]).wait()
            pltpu.make_async_copy(a_src, a_dst, msem.at[3]).wait()

        plsc.subcore_barrier()
        pltpu.sync_copy(
            zeros_hbm, partial.at[pl.ds(s * ROWS_PER_TILE, ROWS_PER_TILE)]
        )
        global_barrier()
        return carry

    lax.fori_loop(0, LAYERS, layer_body, 0)

    foff = 0
    for mlen in MCHUNKS:
        rs = mbase + foff
        Csl = sbuf.at[2] if mlen == CHUNK else sbuf.at[2, pl.ds(0, mlen)]
        pltpu.sync_copy(acc_hbm.at[pl.ds(rs, mlen)], Csl)

        def frow(i, carry4):
            for q in range(EMB // 16):
                sl = pl.ds(q * 16, 16)
                sbuf[2, i, sl] = sbuf[2, i, sl] * 0.25
            return carry4

        lax.fori_loop(0, mlen, frow, 0)
        pltpu.sync_copy(Csl, out_hbm.at[pl.ds(rs, mlen)])
        foff += mlen


def kernel(adj_row, adj_col, adj_values, embedding):
    pad = NNZ_PAD - NNZ
    rows = jnp.concatenate(
        [adj_row.astype(jnp.int32), jnp.zeros((pad,), jnp.int32)]
    ).reshape(NUM_WORKERS, CHUNKS_PER_WORKER, CHUNK)
    cols = jnp.concatenate(
        [adj_col.astype(jnp.int32), jnp.zeros((pad,), jnp.int32)]
    ).reshape(NUM_WORKERS, CHUNKS_PER_WORKER, CHUNK)
    vals = jnp.concatenate(
        [adj_values, jnp.zeros((pad,), jnp.float32)]
    ).reshape(NUM_WORKERS, CHUNKS_PER_WORKER, CHUNK)
    zeros = jnp.zeros((ROWS_PER_TILE, EMB), jnp.float32)

    out, _, _, _ = _hyperconv_sc(rows, cols, vals, embedding, zeros)
    return out


# speedup vs baseline: 1.0286x; 1.0286x over previous
import functools

import jax
import jax.numpy as jnp
from jax import lax
from jax.experimental import pallas as pl
from jax.experimental.pallas import tpu as pltpu
from jax.experimental.pallas import tpu_sc as plsc

N = 16384
EMB = 64
NNZ = 268435
LAYERS = 3

NUM_CORES = 2
NUM_SUBCORES = 16
NUM_WORKERS = NUM_CORES * NUM_SUBCORES
CHUNK = 112
CHUNKS_PER_WORKER = 75
EDGES_PER_WORKER = CHUNK * CHUNKS_PER_WORKER
NNZ_PAD = NUM_WORKERS * EDGES_PER_WORKER
ROWS_PER_TILE = N // NUM_SUBCORES
NBUF = 5
MROWS = N // NUM_WORKERS
HALF = N // NUM_CORES
MCHUNKS = [CHUNK] * (MROWS // CHUNK) + ([MROWS % CHUNK] if MROWS % CHUNK else [])

_mesh = plsc.VectorSubcoreMesh(core_axis_name="c", subcore_axis_name="s")


@functools.partial(
    pl.kernel,
    out_type=[
        jax.ShapeDtypeStruct((N, EMB), jnp.float32),
        jax.ShapeDtypeStruct((N, EMB), jnp.float32),
        jax.ShapeDtypeStruct((N, EMB), jnp.float32),
        jax.ShapeDtypeStruct((NUM_CORES, HALF, EMB), jnp.float32),
    ],
    mesh=_mesh,
    compiler_params=pltpu.CompilerParams(use_tc_tiling_on_sc=False),
    scratch_types=[
        pltpu.VMEM((CHUNKS_PER_WORKER, CHUNK), jnp.int32),
        pltpu.VMEM((CHUNKS_PER_WORKER, CHUNK), jnp.int32),
        pltpu.VMEM((CHUNKS_PER_WORKER, CHUNK), jnp.float32),
        pltpu.VMEM((NBUF, CHUNK, EMB), jnp.float32),
        pltpu.VMEM_SHARED((N, EMB), jnp.float32),
        pltpu.SemaphoreType.DMA((NBUF,)),
        pltpu.SemaphoreType.DMA((NBUF,)),
        pltpu.SemaphoreType.REGULAR,
    ],
)
def _hyperconv_sc(rows_hbm, cols_hbm, vals_hbm, emb_hbm, zeros_hbm,
                  out_hbm, x_hbm, acc_hbm, p_hbm,
                  cols_v, rowi_v, vals_vm, gbuf, partial, gsem, ssem, bar_sem):
    c = lax.axis_index("c")
    s = lax.axis_index("s")
    wid = c * NUM_SUBCORES + s

    def global_barrier():
        plsc.subcore_barrier()

        @pl.when(s == 0)
        def _():
            pl.semaphore_signal(bar_sem, 1, core_index=1 - c)
            pl.semaphore_wait(bar_sem, 1)

        plsc.subcore_barrier()

    pltpu.sync_copy(cols_hbm.at[wid], cols_v)
    pltpu.sync_copy(rows_hbm.at[wid], rowi_v)
    pltpu.sync_copy(vals_hbm.at[wid], vals_vm)

    pltpu.sync_copy(zeros_hbm, partial.at[pl.ds(s * ROWS_PER_TILE, ROWS_PER_TILE)])
    mbase = wid * MROWS
    off = 0
    for mlen in MCHUNKS:
        rs = mbase + off
        src = gbuf.at[0] if mlen == CHUNK else gbuf.at[0, pl.ds(0, mlen)]
        pltpu.sync_copy(emb_hbm.at[pl.ds(rs, mlen)], src)
        pltpu.sync_copy(src, x_hbm.at[pl.ds(rs, mlen)])
        pltpu.sync_copy(src, acc_hbm.at[pl.ds(rs, mlen)])
        off += mlen
    global_barrier()

    def layer_body(layer, carry):
        for b in range(NBUF - 1):
            pltpu.async_copy(x_hbm.at[cols_v.at[b]], gbuf.at[b], gsem.at[b])

        def step(j, b, bp, guard_prev, guard_next):
            pltpu.make_async_copy(
                x_hbm.at[cols_v.at[j]], gbuf.at[b], gsem.at[b]
            ).wait()

            for g in range(CHUNK // 16):
                vv = vals_vm[j, pl.ds(g * 16, 16)]
                for k in range(16):
                    e = g * 16 + k
                    v = vv[k]
                    for q in range(EMB // 16):
                        sl = pl.ds(q * 16, 16)
                        gbuf[b, e, sl] = gbuf[b, e, sl] * v

            pltpu.async_copy(
                gbuf.at[b], partial.at[rowi_v.at[j]], ssem.at[b], add=True
            )

            @pl.when(guard_next)
            def _():
                @pl.when(guard_prev)
                def _():
                    pltpu.make_async_copy(
                        gbuf.at[bp], partial.at[rowi_v.at[j]], ssem.at[bp]
                    ).wait()

                pltpu.async_copy(
                    x_hbm.at[cols_v.at[j + NBUF - 1]], gbuf.at[bp], gsem.at[bp]
                )

        n_outer = CHUNKS_PER_WORKER // NBUF

        def outer_body(o, carry2):
            for b in range(NBUF):
                j = o * NBUF + b
                bp = (b - 1) % NBUF
                guard_prev = jnp.bool_(True) if b != 0 else (o > 0)
                guard_next = j + NBUF - 1 < jnp.int32(CHUNKS_PER_WORKER)
                step(j, b, bp, guard_prev, guard_next)
            return carry2

        lax.fori_loop(0, n_outer, outer_body, 0)
        for t in range(CHUNKS_PER_WORKER - n_outer * NBUF):
            j = n_outer * NBUF + t
            b = j % NBUF
            bp = (b - 1) % NBUF
            step(j, b, bp, jnp.bool_(True),
                 jnp.bool_(j + NBUF - 1 < CHUNKS_PER_WORKER))

        for b in range(NBUF):
            pltpu.make_async_copy(
                gbuf.at[b], partial.at[rowi_v.at[0]], ssem.at[b]
            ).wait()
        plsc.subcore_barrier()

        exp_base = (1 - c) * HALF + s * MROWS
        pltpu.sync_copy(
            partial.at[pl.ds(exp_base, MROWS)],
            p_hbm.at[c, pl.ds(s * MROWS, MROWS)],
        )
        global_barrier()

        moff = 0
        pending = {0: None, 1: None}
        for mi, mlen in enumerate(MCHUNKS):
            par = mi % 2
            ba = 0 if par == 0 else 3
            bc = 2 if par == 0 else 4
            rs = mbase + moff
            ps = mbase - c * HALF + moff
            Asl = gbuf.at[ba] if mlen == CHUNK else gbuf.at[ba, pl.ds(0, mlen)]
            Bsl = gbuf.at[1] if mlen == CHUNK else gbuf.at[1, pl.ds(0, mlen)]
            Csl = gbuf.at[bc] if mlen == CHUNK else gbuf.at[bc, pl.ds(0, mlen)]
            src_a = partial.at[pl.ds(rs, mlen)]
            src_b = p_hbm.at[1 - c, pl.ds(ps, mlen)]
            src_c = acc_hbm.at[pl.ds(rs, mlen)]
            if pending[par] is not None:
                x_src, x_dst, a_src, a_dst = pending[par]
                pltpu.make_async_copy(x_src, x_dst, ssem.at[par]).wait()
                pltpu.make_async_copy(a_src, a_dst, ssem.at[2 + par]).wait()
            pltpu.async_copy(src_a, Asl, gsem.at[0])
            pltpu.async_copy(src_b, Bsl, gsem.at[1])
            pltpu.async_copy(src_c, Csl, gsem.at[2])
            pltpu.make_async_copy(src_a, Asl, gsem.at[0]).wait()
            pltpu.make_async_copy(src_b, Bsl, gsem.at[1]).wait()
            pltpu.make_async_copy(src_c, Csl, gsem.at[2]).wait()

            def mrow(i, carry3):
                for q in range(EMB // 16):
                    sl = pl.ds(q * 16, 16)
                    x = gbuf[ba, i, sl] + gbuf[1, i, sl]
                    gbuf[ba, i, sl] = x
                    gbuf[bc, i, sl] = gbuf[bc, i, sl] + x
                return carry3

            lax.fori_loop(0, mlen, mrow, 0)
            dst_x = x_hbm.at[pl.ds(rs, mlen)]
            dst_a = acc_hbm.at[pl.ds(rs, mlen)]
            pltpu.async_copy(Asl, dst_x, ssem.at[par])
            pltpu.async_copy(Csl, dst_a, ssem.at[2 + par])
            pending[par] = (Asl, dst_x, Csl, dst_a)
            moff += mlen
        for par in (0, 1):
            if pending[par] is not None:
                x_src, x_dst, a_src, a_dst = pending[par]
                pltpu.make_async_copy(x_src, x_dst, ssem.at[par]).wait()
                pltpu.make_async_copy(a_src, a_dst, ssem.at[2 + par]).wait()

        plsc.subcore_barrier()
        pltpu.sync_copy(
            zeros_hbm, partial.at[pl.ds(s * ROWS_PER_TILE, ROWS_PER_TILE)]
        )
        global_barrier()
        return carry

    lax.fori_loop(0, LAYERS, layer_body, 0)

    foff = 0
    for mlen in MCHUNKS:
        rs = mbase + foff
        Csl = gbuf.at[2] if mlen == CHUNK else gbuf.at[2, pl.ds(0, mlen)]
        pltpu.sync_copy(acc_hbm.at[pl.ds(rs, mlen)], Csl)

        def frow(i, carry4):
            for q in range(EMB // 16):
                sl = pl.ds(q * 16, 16)
                gbuf[2, i, sl] = gbuf[2, i, sl] * 0.25
            return carry4

        lax.fori_loop(0, mlen, frow, 0)
        pltpu.sync_copy(Csl, out_hbm.at[pl.ds(rs, mlen)])
        foff += mlen


def kernel(adj_row, adj_col, adj_values, embedding):
    pad = NNZ_PAD - NNZ
    rows = jnp.concatenate(
        [adj_row.astype(jnp.int32), jnp.zeros((pad,), jnp.int32)]
    ).reshape(NUM_WORKERS, CHUNKS_PER_WORKER, CHUNK)
    cols = jnp.concatenate(
        [adj_col.astype(jnp.int32), jnp.zeros((pad,), jnp.int32)]
    ).reshape(NUM_WORKERS, CHUNKS_PER_WORKER, CHUNK)
    vals = jnp.concatenate(
        [adj_values, jnp.zeros((pad,), jnp.float32)]
    ).reshape(NUM_WORKERS, CHUNKS_PER_WORKER, CHUNK)
    zeros = jnp.zeros((ROWS_PER_TILE, EMB), jnp.float32)

    out, _, _, _ = _hyperconv_sc(rows, cols, vals, embedding, zeros)
    return out
